# baseline (speedup 1.0000x reference)
import functools

import jax
import jax.numpy as jnp
from jax import lax
from jax.experimental import pallas as pl
from jax.experimental.pallas import tpu as pltpu
from jax.experimental.pallas import tpu_sc as plsc

_V = 100000
_DEG = 16
_B = 1024
_E = 128
_NC, _NS = 2, 16
_NW = _NC * _NS
_SPT = _B // _NW
_FAN = _DEG * _DEG
_RBLK = 4096
_VP = 102400


def _row_mean_body(t_ref, o_ref):
    ones = jnp.full((_E, 1), 1.0 / _E, jnp.float32)
    s = jax.lax.dot_general(
        t_ref[:], ones, (((1,), (0,)), ((), ())),
        preferred_element_type=jnp.float32)
    o_ref[:] = jnp.transpose(s).reshape(_RBLK)


def _row_means(table):
    return pl.pallas_call(
        _row_mean_body,
        grid=(_VP // _RBLK,),
        in_specs=[pl.BlockSpec((_RBLK, _E), lambda i: (i, 0))],
        out_specs=pl.BlockSpec((_RBLK,), lambda i: (i,)),
        out_shape=jax.ShapeDtypeStruct((_VP,), jnp.float32),
    )(table)


def _sc_body(neigh_hbm, seeds_hbm, rm_hbm, out_hbm,
             seed_v, rows1_v, rows2_v, rm_v, out_v, rm_sem, g_sem):
    wid = lax.axis_index("s") * _NC + lax.axis_index("c")
    base = wid * _SPT

    rm_copy = pltpu.async_copy(rm_hbm, rm_v, rm_sem)

    pltpu.sync_copy(seeds_hbm.at[pl.ds(base, _SPT)], seed_v)
    pltpu.async_copy(neigh_hbm.at[seed_v], rows1_v, g_sem).wait()

    def hop2_fire(j, c):
        idx = rows1_v[j]
        pltpu.async_copy(neigh_hbm.at[idx],
                         rows2_v.at[pl.ds(j * _DEG, _DEG)], g_sem)
        return c
    lax.fori_loop(0, _SPT, hop2_fire, 0)
    pltpu.make_async_copy(neigh_hbm.at[pl.ds(0, _SPT * _DEG)], rows2_v,
                          g_sem).wait()

    rm_copy.wait()

    def mean_round(r, c):
        ids = rows2_v[r]
        out_v[pl.ds(r * _DEG, _DEG)] = plsc.load_gather(rm_v, [ids])
        return c
    lax.fori_loop(0, _SPT * _DEG, mean_round, 0)

    pltpu.sync_copy(out_v, out_hbm.at[pl.ds(base * _FAN, _SPT * _FAN)])


_sc_expand = functools.partial(
    pl.kernel,
    out_type=jax.ShapeDtypeStruct((_B * _FAN,), jnp.float32),
    mesh=plsc.VectorSubcoreMesh(core_axis_name="c", subcore_axis_name="s",
                                num_cores=_NC, num_subcores=_NS),
    compiler_params=pltpu.CompilerParams(needs_layout_passes=False,
                                         use_tc_tiling_on_sc=False),
    scratch_types=[
        pltpu.VMEM((_SPT,), jnp.int32),
        pltpu.VMEM((_SPT, _DEG), jnp.int32),
        pltpu.VMEM((_SPT * _DEG, _DEG), jnp.int32),
        pltpu.VMEM((_VP,), jnp.float32),
        pltpu.VMEM((_SPT * _FAN,), jnp.float32),
        pltpu.SemaphoreType.DMA,
        pltpu.SemaphoreType.DMA,
    ],
)(_sc_body)


def kernel(neighbors, seed_nodes, table):
    rm = _row_means(table)
    out_flat = _sc_expand(neighbors, seed_nodes, rm)
    return out_flat.reshape(_B, _FAN)


# speedup vs baseline: 1.3201x; 1.3201x over previous
import functools

import jax
import jax.numpy as jnp
from jax import lax
from jax.experimental import pallas as pl
from jax.experimental.pallas import tpu as pltpu
from jax.experimental.pallas import tpu_sc as plsc

_V = 100000
_DEG = 16
_B = 1024
_E = 128
_NC, _NS = 2, 16
_NW = _NC * _NS
_SPT = _B // _NW
_FAN = _DEG * _DEG
_RBLK = 4096
_VP = 102400
_NR = _DEG * _V // 16
_SLAB = 128
_RING = 4


def _row_mean_body(t_ref, o_ref):
    ones = jnp.full((_E, 1), 1.0 / _E, jnp.float32)
    s = jax.lax.dot_general(
        t_ref[:], ones, (((1,), (0,)), ((), ())),
        preferred_element_type=jnp.float32)
    o_ref[:] = jnp.transpose(s).reshape(_RBLK)


def _row_means(table):
    return pl.pallas_call(
        _row_mean_body,
        grid=(_VP // _RBLK,),
        in_specs=[pl.BlockSpec((_RBLK, _E), lambda i: (i, 0))],
        out_specs=pl.BlockSpec((_RBLK,), lambda i: (i,)),
        out_shape=jax.ShapeDtypeStruct((_VP,), jnp.float32),
    )(table)


def _expand_body(ntr_hbm, seeds_hbm, ids_hbm,
                 seed_v, ids1_v, idx_v, off_v, ring_v, idsrow_v, s1, s2):
    wid = lax.axis_index("s") * _NC + lax.axis_index("c")
    base = wid * _SPT
    iota = lax.iota(jnp.int32, 16)

    pltpu.sync_copy(seeds_hbm.at[pl.ds(base, _SPT)], seed_v)

    for g in range(2):
        svec = seed_v[pl.ds(g * 16, 16)]
        for k in range(_DEG):
            f = svec + k * _V
            p = (g * _DEG + k) * 16
            idx_v[pl.ds(p, 16)] = f >> 4
            off_v[pl.ds(p, 16)] = f & 15
    for d in range(_RING):
        pltpu.async_copy(ntr_hbm.at[idx_v.at[pl.ds(d * _SLAB, _SLAB)]],
                         ring_v.at[pl.ds(d * _SLAB, _SLAB)], s1)
    for d in range(_RING):
        pltpu.make_async_copy(ntr_hbm.at[pl.ds(0, _SLAB)],
                              ring_v.at[pl.ds(d * _SLAB, _SLAB)], s1).wait()
    for c in range(2 * _DEG):
        off = off_v[pl.ds(c * 16, 16)]
        vals = plsc.load_gather(ring_v, [(c * 16 + iota), off])
        ids1_v[pl.ds(c * 16, 16)] = vals

    def build_q(q, c):
        v1 = ids1_v[pl.ds(q * 16, 16)]
        for h2 in range(_DEG):
            f = v1 + h2 * _V
            p = q * _FAN + h2 * 16
            idx_v[pl.ds(p, 16)] = f >> 4
            off_v[pl.ds(p, 16)] = f & 15
        return c
    lax.fori_loop(0, 2 * _DEG, build_q, 0)

    nslab = 2 * _DEG * _FAN // _SLAB

    def fire(d):
        slot = lax.rem(d, _RING)
        pltpu.async_copy(ntr_hbm.at[idx_v.at[pl.ds(d * _SLAB, _SLAB)]],
                         ring_v.at[pl.ds(slot * _SLAB, _SLAB)], s2)

    def extract(d, c):
        slot = lax.rem(d, _RING)
        e0 = d * _SLAB
        g = e0 >> 12
        for cc in range(_SLAB // 16):
            e = e0 + cc * 16
            h1 = (e >> 8) & 15
            h2 = (e >> 4) & 15
            off = off_v[pl.ds(e, 16)]
            vals = plsc.load_gather(ring_v, [slot * _SLAB + cc * 16 + iota,
                                             off])
            dst = (g * 16 + iota) * _FAN + (h1 * 16 + h2)
            plsc.store_scatter(idsrow_v, [dst], vals)
        return c

    def drain(d):
        slot = lax.rem(d, _RING)
        pltpu.make_async_copy(ntr_hbm.at[pl.ds(0, _SLAB)],
                              ring_v.at[pl.ds(slot * _SLAB, _SLAB)],
                              s2).wait()

    for d in range(_RING):
        fire(d)

    def step(d, c):
        drain(d)
        extract(d, c)
        fire(d + _RING)
        return c
    lax.fori_loop(0, nslab - _RING, step, 0)

    def tail(d, c):
        drain(d)
        extract(d, c)
        return c
    lax.fori_loop(nslab - _RING, nslab, tail, 0)

    pltpu.sync_copy(idsrow_v, ids_hbm.at[pl.ds(base * _FAN, _SPT * _FAN)])


_sc_expand = functools.partial(
    pl.kernel,
    out_type=jax.ShapeDtypeStruct((_B * _FAN,), jnp.int32),
    mesh=plsc.VectorSubcoreMesh(core_axis_name="c", subcore_axis_name="s",
                                num_cores=_NC, num_subcores=_NS),
    compiler_params=pltpu.CompilerParams(needs_layout_passes=False,
                                         use_tc_tiling_on_sc=False),
    scratch_types=[
        pltpu.VMEM((_SPT,), jnp.int32),
        pltpu.VMEM((2 * _DEG * 16,), jnp.int32),
        pltpu.VMEM((_SPT * _FAN,), jnp.int32),
        pltpu.VMEM((_SPT * _FAN,), jnp.int32),
        pltpu.VMEM((_RING * _SLAB, 16), jnp.int32),
        pltpu.VMEM((_SPT * _FAN,), jnp.int32),
        pltpu.SemaphoreType.DMA,
        pltpu.SemaphoreType.DMA,
    ],
)(_expand_body)


def _lookup_body(ids_hbm, rm_hbm, out_hbm, ids_v, rm_v, out_v, rm_sem):
    wid = lax.axis_index("s") * _NC + lax.axis_index("c")
    base = wid * _SPT * _FAN
    rm_copy = pltpu.async_copy(rm_hbm, rm_v, rm_sem)
    pltpu.sync_copy(ids_hbm.at[pl.ds(base, _SPT * _FAN)], ids_v)
    rm_copy.wait()

    def mean_round(r, c):
        ids = ids_v[pl.ds(r * _DEG, _DEG)]
        out_v[pl.ds(r * _DEG, _DEG)] = plsc.load_gather(rm_v, [ids])
        return c
    lax.fori_loop(0, _SPT * _DEG, mean_round, 0)

    pltpu.sync_copy(out_v, out_hbm.at[pl.ds(base, _SPT * _FAN)])


_sc_lookup = functools.partial(
    pl.kernel,
    out_type=jax.ShapeDtypeStruct((_B * _FAN,), jnp.float32),
    mesh=plsc.VectorSubcoreMesh(core_axis_name="c", subcore_axis_name="s",
                                num_cores=_NC, num_subcores=_NS),
    compiler_params=pltpu.CompilerParams(needs_layout_passes=False,
                                         use_tc_tiling_on_sc=False),
    scratch_types=[
        pltpu.VMEM((_SPT * _FAN,), jnp.int32),
        pltpu.VMEM((_VP,), jnp.float32),
        pltpu.VMEM((_SPT * _FAN,), jnp.float32),
        pltpu.SemaphoreType.DMA,
    ],
)(_lookup_body)


def kernel(neighbors, seed_nodes, table):
    ntr = jnp.transpose(neighbors).reshape(_NR, 16)
    ids2 = _sc_expand(ntr, seed_nodes)
    rm = _row_means(table)
    out_flat = _sc_lookup(ids2, rm)
    return out_flat.reshape(_B, _FAN)
